# baseline (speedup 1.0000x reference)
import functools

import jax
import jax.numpy as jnp
from jax import lax
from jax.experimental import pallas as pl
from jax.experimental.pallas import tpu as pltpu
from jax.experimental.pallas import tpu_sc as plsc

_G = 16
_H = 128


def _pad_hi_tc(table):
  A, D = table.shape
  w = D - _H
  bs = 10000

  def body(t_ref, o_ref):
    o_ref[...] = jnp.concatenate(
        [t_ref[...], jnp.zeros((bs, _H - w), jnp.float32)], axis=1)

  return pl.pallas_call(
      body,
      grid=(A // bs,),
      in_specs=[pl.BlockSpec((bs, w), lambda i: (i, 0))],
      out_specs=pl.BlockSpec((bs, _H), lambda i: (i, 0)),
      out_shape=jax.ShapeDtypeStruct((A, _H), jnp.float32),
  )(lax.slice(table, (0, _H), (A, D)))


def _embed_bag_half_sc(C, K, A, src_w):
  info = plsc.get_sparse_core_info()
  nw = info.num_cores * info.num_subcores
  G = _G
  GK = G * K
  n_groups = C // G
  max_w = (n_groups + nw - 1) // nw

  mesh = plsc.VectorSubcoreMesh(core_axis_name="c", subcore_axis_name="s")

  @functools.partial(
      pl.kernel,
      mesh=mesh,
      out_type=jax.ShapeDtypeStruct((C, _H), jnp.float32),
      scratch_types=[
          pltpu.VMEM((max_w * GK,), jnp.int32),
          pltpu.VMEM((GK, _H), jnp.float32),
          pltpu.VMEM((GK, _H), jnp.float32),
          pltpu.VMEM((GK, _H), jnp.float32),
          pltpu.VMEM((GK, _H), jnp.float32),
          pltpu.VMEM((G, _H), jnp.float32),
          pltpu.SemaphoreType.DMA,
          pltpu.SemaphoreType.DMA,
          pltpu.SemaphoreType.DMA,
          pltpu.SemaphoreType.DMA,
      ],
  )
  def bag(idx_hbm, src_hbm, out_hbm, idx_v, buf0, buf1, buf2, buf3,
          acc_v, sem0, sem1, sem2, sem3):
    wid = lax.axis_index("s") * info.num_cores + lax.axis_index("c")
    g0 = wid * n_groups // nw
    g1 = (wid + 1) * n_groups // nw
    n_w = g1 - g0
    pltpu.sync_copy(idx_hbm.at[pl.ds(g0 * GK, max_w * GK)], idx_v)

    def fire(s, buf, sem):
      @pl.when(s < n_w)
      def _():
        ids = idx_v.at[pl.ds(s * GK, GK)]
        pltpu.async_copy(src_hbm.at[ids, pl.ds(0, _H)], buf, sem)

    def wait(buf, sem):
      ids = idx_v.at[pl.ds(0, GK)]
      pltpu.make_async_copy(src_hbm.at[ids, pl.ds(0, _H)], buf, sem).wait()

    def process(s, buf):
      def per_concept(i, _):
        for o in range(0, _H, 16):
          acc = buf[i * K, pl.ds(o, 16)]
          for kk in range(1, K):
            acc = acc + buf[i * K + kk, pl.ds(o, 16)]
          acc_v[i, pl.ds(o, 16)] = acc
        return 0

      lax.fori_loop(0, G, per_concept, 0)
      pltpu.sync_copy(acc_v, out_hbm.at[pl.ds((g0 + s) * G, G)])

    ring = ((buf0, sem0), (buf1, sem1), (buf2, sem2), (buf3, sem3))
    for r, (b, sm) in enumerate(ring):
      fire(r, b, sm)

    def quad(q, _):
      for r, (b, sm) in enumerate(ring):
        s = 4 * q + r
        wait(b, sm)
        process(s, b)
        fire(s + 4, b, sm)
      return 0

    lax.fori_loop(0, n_w // 4, quad, 0)

    base = (n_w // 4) * 4
    for r, (b, sm) in enumerate(ring):
      @pl.when(base + r < n_w)
      def _(b=b, sm=sm, r=r):
        wait(b, sm)
        process(base + r, b)

  return bag


def _matmul_softmax_tc(x, h_lo, h_hi, bb):
  B, Dp = x.shape
  C = h_lo.shape[0]

  def body(x_ref, lo_ref, hi_ref, o_ref):
    logits = lax.dot_general(
        x_ref[:, :_H], lo_ref[...], (((1,), (1,)), ((), ())),
        preferred_element_type=jnp.float32)
    logits = logits + lax.dot_general(
        x_ref[:, _H:], hi_ref[...], (((1,), (1,)), ((), ())),
        preferred_element_type=jnp.float32)
    m = jnp.max(logits, axis=1, keepdims=True)
    e = jnp.exp(logits - m)
    o_ref[...] = e * (1.0 / jnp.sum(e, axis=1, keepdims=True))

  return pl.pallas_call(
      body,
      grid=(B // bb,),
      in_specs=[
          pl.BlockSpec((bb, Dp), lambda i: (i, 0)),
          pl.BlockSpec((C, _H), lambda i: (0, 0)),
          pl.BlockSpec((C, _H), lambda i: (0, 0)),
      ],
      out_specs=pl.BlockSpec((bb, C), lambda i: (i, 0)),
      out_shape=jax.ShapeDtypeStruct((B, C), jnp.float32),
  )(x, h_lo, h_hi)


def kernel(x, ancestor_idx, table):
  C, K = ancestor_idx.shape
  A, D = table.shape
  idx = ancestor_idx.astype(jnp.int32).reshape(-1)
  h_lo = _embed_bag_half_sc(C, K, A, D)(idx, table)
  t_hi = _pad_hi_tc(table)
  h_hi = _embed_bag_half_sc(C, K, A, _H)(idx, t_hi)
  x_p = jnp.pad(x, ((0, 0), (0, 2 * _H - D)))
  return _matmul_softmax_tc(x_p, h_lo, h_hi, 128)


# speedup vs baseline: 1.0107x; 1.0107x over previous
import functools

import jax
import jax.numpy as jnp
from jax import lax
from jax.experimental import pallas as pl
from jax.experimental.pallas import tpu as pltpu
from jax.experimental.pallas import tpu_sc as plsc

_G = 16
_H = 128


def _pad_hi_tc(table):
  A, D = table.shape
  w = D - _H
  bs = 10000

  def body(t_ref, o_ref):
    o_ref[...] = jnp.concatenate(
        [t_ref[...], jnp.zeros((bs, _H - w), jnp.float32)], axis=1)

  return pl.pallas_call(
      body,
      grid=(A // bs,),
      in_specs=[pl.BlockSpec((bs, w), lambda i: (i, 0))],
      out_specs=pl.BlockSpec((bs, _H), lambda i: (i, 0)),
      out_shape=jax.ShapeDtypeStruct((A, _H), jnp.float32),
  )(lax.slice(table, (0, _H), (A, D)))


def _embed_bag_half_sc(C, K, A, src_w):
  info = plsc.get_sparse_core_info()
  nw = info.num_cores * info.num_subcores
  G = _G
  GK = G * K
  n_groups = C // G
  max_w = (n_groups + nw - 1) // nw

  mesh = plsc.VectorSubcoreMesh(core_axis_name="c", subcore_axis_name="s")

  @functools.partial(
      pl.kernel,
      mesh=mesh,
      out_type=jax.ShapeDtypeStruct((C, _H), jnp.float32),
      scratch_types=[
          pltpu.VMEM((max_w * GK,), jnp.int32),
          pltpu.VMEM((GK, _H), jnp.float32),
          pltpu.VMEM((GK, _H), jnp.float32),
          pltpu.VMEM((G, _H), jnp.float32),
          pltpu.SemaphoreType.DMA,
          pltpu.SemaphoreType.DMA,
      ],
  )
  def bag(idx_hbm, src_hbm, out_hbm, idx_v, buf0, buf1, acc_v, sem0, sem1):
    wid = lax.axis_index("s") * info.num_cores + lax.axis_index("c")
    g0 = wid * n_groups // nw
    g1 = (wid + 1) * n_groups // nw
    n_w = g1 - g0
    pltpu.sync_copy(idx_hbm.at[pl.ds(g0 * GK, max_w * GK)], idx_v)

    def fire(s, buf, sem):
      @pl.when(s < n_w)
      def _():
        ids = idx_v.at[pl.ds(s * GK, GK)]
        pltpu.async_copy(src_hbm.at[ids, pl.ds(0, _H)], buf, sem)

    def wait(buf, sem):
      ids = idx_v.at[pl.ds(0, GK)]
      pltpu.make_async_copy(src_hbm.at[ids, pl.ds(0, _H)], buf, sem).wait()

    def process(s, buf):
      def per_concept(i, _):
        for o in range(0, _H, 16):
          acc = buf[i * K, pl.ds(o, 16)]
          for kk in range(1, K):
            acc = acc + buf[i * K + kk, pl.ds(o, 16)]
          acc_v[i, pl.ds(o, 16)] = acc
        return 0

      lax.fori_loop(0, G, per_concept, 0)
      pltpu.sync_copy(acc_v, out_hbm.at[pl.ds((g0 + s) * G, G)])

    fire(0, buf0, sem0)
    fire(1, buf1, sem1)

    def pair(p, _):
      s0 = 2 * p
      wait(buf0, sem0)
      process(s0, buf0)
      fire(s0 + 2, buf0, sem0)

      s1 = 2 * p + 1
      wait(buf1, sem1)
      process(s1, buf1)
      fire(s1 + 2, buf1, sem1)
      return 0

    lax.fori_loop(0, n_w // 2, pair, 0)

    @pl.when(n_w % 2 == 1)
    def _():
      wait(buf0, sem0)
      process(n_w - 1, buf0)

  return bag


def _matmul_softmax_tc(x, h_lo, h_hi, bb):
  B, D = x.shape
  C = h_lo.shape[0]

  def body(x_ref, lo_ref, hi_ref, o_ref):
    x_hi = jnp.concatenate(
        [x_ref[:, _H:], jnp.zeros((bb, 2 * _H - D), jnp.float32)], axis=1)
    logits = lax.dot_general(
        x_ref[:, :_H], lo_ref[...], (((1,), (1,)), ((), ())),
        preferred_element_type=jnp.float32)
    logits = logits + lax.dot_general(
        x_hi, hi_ref[...], (((1,), (1,)), ((), ())),
        preferred_element_type=jnp.float32)
    m = jnp.max(logits, axis=1, keepdims=True)
    e = jnp.exp(logits - m)
    o_ref[...] = e * (1.0 / jnp.sum(e, axis=1, keepdims=True))

  return pl.pallas_call(
      body,
      grid=(B // bb,),
      in_specs=[
          pl.BlockSpec((bb, D), lambda i: (i, 0)),
          pl.BlockSpec((C, _H), lambda i: (0, 0)),
          pl.BlockSpec((C, _H), lambda i: (0, 0)),
      ],
      out_specs=pl.BlockSpec((bb, C), lambda i: (i, 0)),
      out_shape=jax.ShapeDtypeStruct((B, C), jnp.float32),
  )(x, h_lo, h_hi)


def kernel(x, ancestor_idx, table):
  C, K = ancestor_idx.shape
  A, D = table.shape
  idx = ancestor_idx.astype(jnp.int32).reshape(-1)
  h_lo = _embed_bag_half_sc(C, K, A, D)(idx, table)
  t_hi = _pad_hi_tc(table)
  h_hi = _embed_bag_half_sc(C, K, A, _H)(idx, t_hi)
  return _matmul_softmax_tc(x, h_lo, h_hi, 128)
